# Initial kernel scaffold; baseline (speedup 1.0000x reference)
import jax
import jax.numpy as jnp
from jax.experimental import pallas as pl


def kernel(hidden_states, Wr, Wg, Wu, Wd, Sg, Su, Sd):
    raise NotImplementedError("write your pallas kernel here")


# speedup vs baseline: 2.0120x; 2.0120x over previous
import functools

import jax
import jax.numpy as jnp
from jax.experimental import pallas as pl
from jax.experimental.pallas import tpu as pltpu

T, D, F, E = 2048, 1024, 512, 8


def _silu(x):
    return x * jax.nn.sigmoid(x)


def _fused_body(x_ref, wr_ref, wg_ref, wu_ref, wd_ref, sg_ref, su_ref, sd_ref,
                out_ref, comb_ref):
    e = pl.program_id(0)

    @pl.when(e == 0)
    def _init():
        x = x_ref[...]
        logits = jnp.dot(x, wr_ref[...], preferred_element_type=jnp.float32)
        idx = jnp.argmax(logits, axis=1)
        w = jax.nn.sigmoid(jnp.max(logits, axis=1))
        onehot = (jax.lax.broadcasted_iota(jnp.int32, (T, E), 1)
                  == idx[:, None]).astype(jnp.float32)
        comb_ref[...] = onehot * w[:, None]
        g = jnp.dot(x, sg_ref[...], preferred_element_type=jnp.float32)
        u = jnp.dot(x, su_ref[...], preferred_element_type=jnp.float32)
        a = _silu(g) * u
        out_ref[...] = jnp.dot(a, sd_ref[...], preferred_element_type=jnp.float32)

    x = x_ref[...]
    g = jnp.dot(x, wg_ref[0], preferred_element_type=jnp.float32)
    u = jnp.dot(x, wu_ref[0], preferred_element_type=jnp.float32)
    a = _silu(g) * u
    y = jnp.dot(a, wd_ref[0], preferred_element_type=jnp.float32)
    comb = comb_ref[...]
    col = jnp.sum(
        jnp.where(jax.lax.broadcasted_iota(jnp.int32, (T, E), 1) == e, comb, 0.0),
        axis=1, keepdims=True)
    out_ref[...] += col * y


@jax.jit
def kernel(hidden_states, Wr, Wg, Wu, Wd, Sg, Su, Sd):
    out = pl.pallas_call(
        _fused_body,
        grid=(E,),
        in_specs=[
            pl.BlockSpec((T, D), lambda e: (0, 0)),
            pl.BlockSpec((D, E), lambda e: (0, 0)),
            pl.BlockSpec((1, D, F), lambda e: (e, 0, 0)),
            pl.BlockSpec((1, D, F), lambda e: (e, 0, 0)),
            pl.BlockSpec((1, F, D), lambda e: (e, 0, 0)),
            pl.BlockSpec((D, F), lambda e: (0, 0)),
            pl.BlockSpec((D, F), lambda e: (0, 0)),
            pl.BlockSpec((F, D), lambda e: (0, 0)),
        ],
        out_specs=pl.BlockSpec((T, D), lambda e: (0, 0)),
        out_shape=jax.ShapeDtypeStruct((T, D), jnp.float32),
        scratch_shapes=[pltpu.VMEM((T, E), jnp.float32)],
    )(hidden_states, Wr, Wg, Wu, Wd, Sg, Su, Sd)
    return out
